# baseline (speedup 1.0000x reference)
import functools

import jax
import jax.numpy as jnp
from jax import lax
from jax.experimental import pallas as pl
from jax.experimental.pallas import tpu as pltpu
from jax.experimental.pallas import tpu_sc as plsc

_B = 4096
_S = 50
_D = 64
_NC = 2
_NS = 16
_NW = _NC * _NS
_L = 128
_JC = 5
_N_CHUNKS = _S // _JC

_mesh = plsc.VectorSubcoreMesh(core_axis_name="c", subcore_axis_name="s")


@functools.partial(
    pl.kernel,
    mesh=_mesh,
    out_type=jax.ShapeDtypeStruct((_S, _D // 8, _NW, 8 * _L), jnp.float32),
    scratch_types=(
        [pltpu.VMEM((_L,), jnp.int32) for _ in range(_JC)]
        + [pltpu.VMEM((_L, _D), jnp.float32) for _ in range(_JC)]
        + [pltpu.VMEM((_D * _L,), jnp.float32)]
        + [pltpu.SemaphoreType.DMA for _ in range(3)]
    ),
    compiler_params=pltpu.CompilerParams(
        use_tc_tiling_on_sc=False, needs_layout_passes=False),
)
def _gather_t(idx_hbm, table_hbm, out_hbm, *refs):
    idx_bufs = refs[:_JC]
    row_bufs = refs[_JC:2 * _JC]
    tj = refs[2 * _JC]
    sem_i, sem_g, sem_s = refs[2 * _JC + 1:]

    wid = lax.axis_index("s") * _NC + lax.axis_index("c")
    s0 = wid * _L

    iota = lax.iota(jnp.int32, 16)

    def chunk_body(c, carry):
        icopies = [
            pltpu.async_copy(
                idx_hbm.at[c * _JC + jj, pl.ds(s0, _L)], idx_bufs[jj], sem_i)
            for jj in range(_JC)
        ]
        gcopies = []
        for jj in range(_JC):
            icopies[jj].wait()
            gcopies.append(pltpu.async_copy(
                table_hbm.at[idx_bufs[jj]], row_bufs[jj], sem_g))
        rowvs = [iota + (so * 16) for so in range(_L // 16)]
        for jj in range(_JC):
            gcopies[jj].wait()
            j = c * _JC + jj

            @plsc.parallel_loop(0, _D, unroll=8)
            def d_body(d, jj=jj):
                col = jnp.broadcast_to(d, (16,))
                off = (d // 8) * (8 * _L) + (d % 8) * _L
                for so in range(_L // 16):
                    v = plsc.load_gather(row_bufs[jj], [rowvs[so], col])
                    tj[pl.ds(off + so * 16, 16)] = v
            scopies = [
                pltpu.async_copy(
                    tj.at[pl.ds(dh * 8 * _L, 8 * _L)],
                    out_hbm.at[j, dh, wid], sem_s)
                for dh in range(8)
            ]
            for sc in scopies:
                sc.wait()
        return carry

    lax.fori_loop(0, _N_CHUNKS, chunk_body, 0)


def kernel(indices, W):
    idx_t = indices.T.astype(jnp.int32)
    out4 = _gather_t(idx_t, W)
    out = (out4.reshape(_S, _D // 8, _NW, 8, _L)
           .transpose(2, 4, 0, 1, 3)
           .reshape(_B, _S, _D))
    return out


# speedup vs baseline: 1.0324x; 1.0324x over previous
import functools

import jax
import jax.numpy as jnp
from jax import lax
from jax.experimental import pallas as pl
from jax.experimental.pallas import tpu as pltpu
from jax.experimental.pallas import tpu_sc as plsc

_B = 4096
_S = 50
_D = 64
_NC = 2
_NS = 16
_NW = _NC * _NS
_L = 128
_JC = 5
_N_CHUNKS = _S // _JC

_mesh = plsc.VectorSubcoreMesh(core_axis_name="c", subcore_axis_name="s")


@functools.partial(
    pl.kernel,
    mesh=_mesh,
    out_type=jax.ShapeDtypeStruct((_S, _D // 8, _NW, 8 * _L), jnp.float32),
    scratch_types=(
        [pltpu.VMEM((_L,), jnp.int32) for _ in range(_JC)]
        + [pltpu.VMEM((_L, _D), jnp.float32) for _ in range(_JC)]
        + [pltpu.VMEM((_D * _L,), jnp.float32)]
        + [pltpu.SemaphoreType.DMA for _ in range(3)]
    ),
    compiler_params=pltpu.CompilerParams(
        use_tc_tiling_on_sc=False, needs_layout_passes=False),
)
def _gather_t(idx_hbm, table_hbm, out_hbm, *refs):
    idx_bufs = refs[:_JC]
    row_bufs = refs[_JC:2 * _JC]
    tj = refs[2 * _JC]
    sem_i, sem_g, sem_s = refs[2 * _JC + 1:]

    wid = lax.axis_index("s") * _NC + lax.axis_index("c")
    s0 = wid * _L

    iota = lax.iota(jnp.int32, 16)

    def chunk_body(c, carry):
        icopies = [
            pltpu.async_copy(
                idx_hbm.at[c * _JC + jj, pl.ds(s0, _L)], idx_bufs[jj], sem_i)
            for jj in range(_JC)
        ]
        gcopies = []
        for jj in range(_JC):
            icopies[jj].wait()
            gcopies.append(pltpu.async_copy(
                table_hbm.at[idx_bufs[jj]], row_bufs[jj], sem_g))
        rowvs = [iota + (so * 16) for so in range(_L // 16)]
        for jj in range(_JC):
            gcopies[jj].wait()
            j = c * _JC + jj

            @plsc.parallel_loop(0, _D, unroll=2)
            def d_body(d, jj=jj):
                col = jnp.broadcast_to(d, (16,))
                off = (d // 8) * (8 * _L) + (d % 8) * _L
                for so in range(_L // 16):
                    v = plsc.load_gather(row_bufs[jj], [rowvs[so], col])
                    tj[pl.ds(off + so * 16, 16)] = v
            scopies = [
                pltpu.async_copy(
                    tj.at[pl.ds(dh * 8 * _L, 8 * _L)],
                    out_hbm.at[j, dh, wid], sem_s)
                for dh in range(8)
            ]
            for sc in scopies:
                sc.wait()
        return carry

    lax.fori_loop(0, _N_CHUNKS, chunk_body, 0)


def kernel(indices, W):
    idx_t = indices.T.astype(jnp.int32)
    out4 = _gather_t(idx_t, W)
    out = (out4.reshape(_S, _D // 8, _NW, 8, _L)
           .transpose(2, 4, 0, 1, 3)
           .reshape(_B, _S, _D))
    return out


# speedup vs baseline: 2.1439x; 2.0767x over previous
import functools

import jax
import jax.numpy as jnp
from jax import lax
from jax.experimental import pallas as pl
from jax.experimental.pallas import tpu as pltpu
from jax.experimental.pallas import tpu_sc as plsc

_B = 4096
_S = 50
_D = 64
_NC = 2
_NS = 16
_NW = _NC * _NS
_L = 128
_LP = _L + 1
_JC = 5
_N_CHUNKS = _S // _JC

_mesh = plsc.VectorSubcoreMesh(core_axis_name="c", subcore_axis_name="s")


@functools.partial(
    pl.kernel,
    mesh=_mesh,
    out_type=jax.ShapeDtypeStruct((_S, _D // 8, _NW, 8, _L), jnp.float32),
    scratch_types=(
        [pltpu.VMEM((_L,), jnp.int32) for _ in range(_JC)]
        + [pltpu.VMEM((_L, _D), jnp.float32) for _ in range(_JC)]
        + [pltpu.VMEM((_D, _LP), jnp.float32)]
        + [pltpu.SemaphoreType.DMA for _ in range(3)]
    ),
    compiler_params=pltpu.CompilerParams(
        use_tc_tiling_on_sc=False, needs_layout_passes=False),
)
def _gather_t(idx_hbm, table_hbm, out_hbm, *refs):
    idx_bufs = refs[:_JC]
    row_bufs = refs[_JC:2 * _JC]
    tj = refs[2 * _JC]
    sem_i, sem_g, sem_s = refs[2 * _JC + 1:]

    wid = lax.axis_index("s") * _NC + lax.axis_index("c")
    s0 = wid * _L

    iota = lax.iota(jnp.int32, 16)
    drow = [iota + d0 for d0 in range(0, _D, 16)]

    def chunk_body(c, carry):
        icopies = [
            pltpu.async_copy(
                idx_hbm.at[c * _JC + jj, pl.ds(s0, _L)], idx_bufs[jj], sem_i)
            for jj in range(_JC)
        ]
        gcopies = []
        for jj in range(_JC):
            icopies[jj].wait()
            gcopies.append(pltpu.async_copy(
                table_hbm.at[idx_bufs[jj]], row_bufs[jj], sem_g))
        for jj in range(_JC):
            gcopies[jj].wait()
            j = c * _JC + jj

            @plsc.parallel_loop(0, _L, unroll=4)
            def sl_body(sl, jj=jj):
                lane = jnp.broadcast_to(sl, (16,))
                for k in range(_D // 16):
                    v = row_bufs[jj][sl, pl.ds(k * 16, 16)]
                    plsc.store_scatter(tj, [drow[k], lane], v)

            scopies = [
                pltpu.async_copy(
                    tj.at[pl.ds(dh * 8, 8), pl.ds(0, _L)],
                    out_hbm.at[j, dh, wid], sem_s)
                for dh in range(8)
            ]
            for sc in scopies:
                sc.wait()
        return carry

    lax.fori_loop(0, _N_CHUNKS, chunk_body, 0)


def kernel(indices, W):
    idx_t = indices.T.astype(jnp.int32)
    out5 = _gather_t(idx_t, W)
    out = out5.transpose(2, 4, 0, 1, 3).reshape(_B, _S, _D)
    return out
